# Initial kernel scaffold; baseline (speedup 1.0000x reference)
import jax
import jax.numpy as jnp
from jax.experimental import pallas as pl


def kernel(inputs, training, centers, wA, bA, w1, b1, w2, b2, wO, bO):
    raise NotImplementedError("write your pallas kernel here")


# speedup vs baseline: 1.6345x; 1.6345x over previous
import functools

import jax
import jax.numpy as jnp
import numpy as np
from jax.experimental import pallas as pl
from jax.experimental.pallas import tpu as pltpu

_NC = 6
_SIGMA = 1.0
_DN = ('NCDHW', 'OIDHW', 'NCDHW')


def _mk_mask(mtype):
    m = np.zeros((3, 3, 3), np.float32)
    m[:1] = 1.0
    m[1, :1] = 1.0
    m[1, 1, :1] = 1.0
    if mtype == 'B':
        m[1, 1, 1] = 1.0
    return jnp.asarray(m)[None, None]

_MA = _mk_mask('A')
_MB = _mk_mask('B')


def _quant_body(c_ref, x_ref, qbar_ref, sym_ref):
    x = x_ref[...]
    ds = []
    for l in range(_NC):
        ds.append(jnp.square(x - c_ref[l]))
    best = ds[0]
    idx = jnp.zeros(x.shape, jnp.int32)
    for l in range(1, _NC):
        cond = ds[l] < best
        idx = jnp.where(cond, jnp.int32(l), idx)
        best = jnp.where(cond, ds[l], best)
    s = jnp.zeros_like(x)
    qs = jnp.zeros_like(x)
    qh = jnp.zeros_like(x)
    for l in range(_NC):
        e = jnp.exp(_SIGMA * (best - ds[l]))
        s = s + e
        qs = qs + e * c_ref[l]
        qh = jnp.where(idx == l, c_ref[l], qh)
    qsoft = qs / s
    qbar_ref[...] = qsoft + (qh - qsoft)
    sym_ref[...] = idx


def _quantize(x, centers):
    n, h, w, c = x.shape
    xf = x.reshape(n * h * w, c)
    qbar, sym = pl.pallas_call(
        _quant_body,
        out_shape=(
            jax.ShapeDtypeStruct(xf.shape, jnp.float32),
            jax.ShapeDtypeStruct(xf.shape, jnp.int32),
        ),
        in_specs=[pl.BlockSpec(memory_space=pltpu.SMEM),
                  pl.BlockSpec(memory_space=pltpu.VMEM)],
        out_specs=(pl.BlockSpec(memory_space=pltpu.VMEM),
                   pl.BlockSpec(memory_space=pltpu.VMEM)),
    )(centers, xf)
    return qbar.reshape(x.shape), sym.reshape(x.shape)


def _conv(x, w, b, padding):
    y = jax.lax.conv_general_dilated(x, w, (1, 1, 1), padding,
                                     dimension_numbers=_DN)
    return y + b[None, :, None, None, None]


def kernel(inputs, training, centers, wA, bA, w1, b1, w2, b2, wO, bO):
    qbar, symbols = _quantize(inputs, centers)
    pc_in = jnp.transpose(qbar, (0, 3, 1, 2))
    sym = jnp.transpose(symbols, (0, 3, 1, 2))
    pad_value = centers[0]
    x = pc_in[:, None]
    xp = jnp.pad(x, ((0, 0), (0, 0), (1, 1), (1, 1), (1, 1)), mode='constant',
                 constant_values=pad_value)
    h = jax.nn.relu(_conv(xp, wA * _MA.astype(wA.dtype), bA, 'VALID'))
    r = jax.nn.relu(_conv(h, w1 * _MB.astype(w1.dtype), b1, 'SAME'))
    r = _conv(r, w2 * _MB.astype(w2.dtype), b2, 'SAME')
    h = jax.nn.relu(h + r)
    logits = _conv(h, wO * _MB.astype(wO.dtype), bO, 'SAME')
    logp = jax.nn.log_softmax(logits, axis=1)
    nll = -jnp.take_along_axis(logp, sym[:, None].astype(jnp.int32), axis=1)[:, 0]
    bc = nll / jnp.log(2.0)
    return (qbar, jnp.transpose(bc, (0, 2, 3, 1)))


# speedup vs baseline: 15.9948x; 9.7860x over previous
import jax
import jax.numpy as jnp
import numpy as np
from jax.experimental import pallas as pl
from jax.experimental.pallas import tpu as pltpu

_NC = 6
_SIGMA = 1.0
_D = 96
_PLANE = 34 * 34
_G = 8
_NA = _D // _G
_NB = _G * _PLANE
_NP = _D + 3
_L = _NP * _PLANE

_TAPS_A = [(0, kh, kw) for kh in range(3) for kw in range(3)] + \
          [(1, 0, 0), (1, 0, 1), (1, 0, 2), (1, 1, 0)]
_TAPS_B = _TAPS_A + [(1, 1, 1)]
_INV_LN2 = float(1.0 / np.log(2.0))


def _delta(kd, kh, kw):
    return (1 + kd) * _PLANE + (kh - 1) * 34 + (kw - 1)


def _quant_body(c_ref, x_ref, qbar_ref, sym_ref):
    x = x_ref[...]
    ds = [jnp.square(x - c_ref[l]) for l in range(_NC)]
    best = ds[0]
    idx = jnp.zeros(x.shape, jnp.int32)
    for l in range(1, _NC):
        cond = ds[l] < best
        idx = jnp.where(cond, jnp.int32(l), idx)
        best = jnp.where(cond, ds[l], best)
    s = jnp.zeros_like(x)
    qs = jnp.zeros_like(x)
    qh = jnp.zeros_like(x)
    for l in range(_NC):
        e = jnp.exp(_SIGMA * (best - ds[l]))
        s = s + e
        qs = qs + e * c_ref[l]
        qh = jnp.where(idx == l, c_ref[l], qh)
    qsoft = qs / s
    qbar_ref[...] = qsoft + (qh - qsoft)
    sym_ref[...] = idx


def _quantize(x, centers):
    n, h, w, c = x.shape
    xf = x.reshape(n * h * w, c)
    qbar, sym = pl.pallas_call(
        _quant_body,
        out_shape=(
            jax.ShapeDtypeStruct(xf.shape, jnp.float32),
            jax.ShapeDtypeStruct(xf.shape, jnp.int32),
        ),
        in_specs=[pl.BlockSpec(memory_space=pltpu.SMEM),
                  pl.BlockSpec(memory_space=pltpu.VMEM)],
        out_specs=(pl.BlockSpec(memory_space=pltpu.VMEM),
                   pl.BlockSpec(memory_space=pltpu.VMEM)),
    )(centers, xf)
    return qbar.reshape(x.shape), sym.reshape(x.shape)


def _pcnn_body(xp_ref, sym_ref, mask_ref, wa_ref, ba_ref, w1_ref, b1_ref,
               w2_ref, b2_ref, wo_ref, bo_ref, bc_ref, ha, r1, xa, xb):
    f32 = jnp.float32
    mask = mask_ref[...]

    z2 = jnp.zeros((24, 2 * _PLANE), f32)
    ha[:, : 2 * _PLANE] = z2
    r1[:, : 2 * _PLANE] = z2
    zg = jnp.zeros((24, _PLANE), f32)
    ha[:, (_NP - 1) * _PLANE:] = zg
    r1[:, (_NP - 1) * _PLANE:] = zg
    xa[13:16, :] = jnp.zeros((3, _NB), f32)

    def conva(a):
        base = a * _NB
        for ti, (kd, kh, kw) in enumerate(_TAPS_A):
            s = base + _delta(kd, kh, kw)
            xa[pl.ds(ti, 1), :] = xp_ref[0, :, pl.ds(s, _NB)]
        y = jnp.dot(wa_ref[...], xa[...], preferred_element_type=f32)
        blk = jnp.maximum(y + ba_ref[...], 0.0) * mask
        ha[:, pl.ds(base + 2 * _PLANE, _NB)] = blk

    def im2col(src, a):
        base = a * _NB
        for ti, (kd, kh, kw) in enumerate(_TAPS_B):
            s = base + _delta(kd, kh, kw)
            xb[pl.ds(24 * ti, 24), :] = src[:, pl.ds(s, _NB)]

    def layer1(a):
        im2col(ha, a)
        y = jnp.dot(w1_ref[...], xb[...], preferred_element_type=f32)
        blk = jnp.maximum(y + b1_ref[...], 0.0) * mask
        r1[:, pl.ds(a * _NB + 2 * _PLANE, _NB)] = blk

    def layer2(a):
        im2col(r1, a)
        y = jnp.dot(w2_ref[...], xb[...], preferred_element_type=f32)
        sl = pl.ds(a * _NB + 2 * _PLANE, _NB)
        h = jnp.maximum(ha[:, sl] + y + b2_ref[...], 0.0) * mask
        ha[:, sl] = h

    def layero(a):
        im2col(ha, a)
        z = jnp.dot(wo_ref[...], xb[...], preferred_element_type=f32)
        z = z + bo_ref[...]
        m = jnp.max(z, axis=0, keepdims=True)
        lse = m + jnp.log(jnp.sum(jnp.exp(z - m), axis=0, keepdims=True))
        logp = z - lse
        sym = sym_ref[0, :, pl.ds(a * _NB, _NB)]
        bc = jnp.zeros((1, _NB), f32)
        for l in range(_NC):
            bc = jnp.where(sym == l, -logp[l:l + 1, :], bc)
        bc_ref[0, :, pl.ds(a * _NB, _NB)] = bc * _INV_LN2

    for a in range(_NA):
        conva(a)
    for a in range(_NA):
        layer1(a)
    for a in range(_NA):
        layer2(a)
    for a in range(_NA):
        layero(a)


def _pcnn(xp, symf, wam, ba, w1m, b1, w2m, b2, wom, bo):
    n = xp.shape[0]
    pos = jnp.arange(_NB, dtype=jnp.int32) % _PLANE
    hp = pos // 34
    wp = pos % 34
    mask = ((hp >= 1) & (hp < 33) & (wp >= 1) & (wp < 33))
    mask = mask.astype(jnp.float32)[None, :]
    in_specs = [
            pl.BlockSpec((1, 1, _L), lambda i: (i, 0, 0)),
            pl.BlockSpec((1, 1, _D * _PLANE), lambda i: (i, 0, 0)),
            pl.BlockSpec((1, _NB), lambda i: (0, 0)),
            pl.BlockSpec((24, 16), lambda i: (0, 0)),
            pl.BlockSpec((24, 1), lambda i: (0, 0)),
            pl.BlockSpec((24, 336), lambda i: (0, 0)),
            pl.BlockSpec((24, 1), lambda i: (0, 0)),
            pl.BlockSpec((24, 336), lambda i: (0, 0)),
            pl.BlockSpec((24, 1), lambda i: (0, 0)),
            pl.BlockSpec((8, 336), lambda i: (0, 0)),
            pl.BlockSpec((8, 1), lambda i: (0, 0)),
        ]
    return pl.pallas_call(
        _pcnn_body,
        grid=(n,),
        in_specs=in_specs,
        out_specs=pl.BlockSpec((1, 1, _D * _PLANE), lambda i: (i, 0, 0)),
        out_shape=jax.ShapeDtypeStruct((n, 1, _D * _PLANE), jnp.float32),
        scratch_shapes=[
            pltpu.VMEM((24, _L), jnp.float32),
            pltpu.VMEM((24, _L), jnp.float32),
            pltpu.VMEM((16, _NB), jnp.float32),
            pltpu.VMEM((336, _NB), jnp.float32),
        ],
        compiler_params=pltpu.CompilerParams(
            dimension_semantics=("arbitrary",)),
    )(xp, symf, mask, wam, ba, w1m, b1, w2m, b2, wom, bo)


def kernel(inputs, training, centers, wA, bA, w1, b1, w2, b2, wO, bO):
    n = inputs.shape[0]
    qbar, sym = _quantize(inputs, centers)
    c0 = centers[0]

    pc = jnp.transpose(qbar, (0, 3, 1, 2))
    pcp = jnp.pad(pc, ((0, 0), (0, 0), (1, 1), (1, 1)), constant_values=c0)
    padp = jnp.broadcast_to(c0, (n, 1, 34, 34))
    guard = jnp.zeros((n, 1, 34, 34), jnp.float32)
    xp = jnp.concatenate([guard, padp, pcp, guard], axis=1)
    xp = xp.reshape(n, 1, _L)

    symt = jnp.transpose(sym, (0, 3, 1, 2))
    symp = jnp.pad(symt, ((0, 0), (0, 0), (1, 1), (1, 1)))
    symf = symp.reshape(n, 1, _D * _PLANE)

    wam = jnp.stack([wA[:, 0, kd, kh, kw] for kd, kh, kw in _TAPS_A], axis=1)
    wam = jnp.pad(wam, ((0, 0), (0, 3)))
    w1m = jnp.stack([w1[:, :, kd, kh, kw] for kd, kh, kw in _TAPS_B],
                    axis=1).reshape(24, 336)
    w2m = jnp.stack([w2[:, :, kd, kh, kw] for kd, kh, kw in _TAPS_B],
                    axis=1).reshape(24, 336)
    wom = jnp.stack([wO[:, :, kd, kh, kw] for kd, kh, kw in _TAPS_B],
                    axis=1).reshape(_NC, 336)
    wom = jnp.pad(wom, ((0, 2), (0, 0)))
    bop = jnp.concatenate([bO, jnp.full((2,), -1e30, jnp.float32)])

    bc = _pcnn(xp, symf, wam, bA[:, None], w1m, b1[:, None], w2m, b2[:, None],
               wom, bop[:, None])
    bc = bc.reshape(n, _D, 34, 34)[:, :, 1:33, 1:33]
    return (qbar, jnp.transpose(bc, (0, 2, 3, 1)))


# speedup vs baseline: 21.8680x; 1.3672x over previous
import jax
import jax.numpy as jnp
import numpy as np
from jax.experimental import pallas as pl
from jax.experimental.pallas import tpu as pltpu

_NC = 6
_SIGMA = 1.0
_D = 96
_PLANE = 34 * 34
_G = 8
_NA = _D // _G
_NB = _G * _PLANE
_NP = _D + 3
_L = _NP * _PLANE

_TAPS_A = [(0, kh, kw) for kh in range(3) for kw in range(3)] + \
          [(1, 0, 0), (1, 0, 1), (1, 0, 2), (1, 1, 0)]
_TAPS_B = _TAPS_A + [(1, 1, 1)]
_INV_LN2 = float(1.0 / np.log(2.0))


def _delta(kd, kh, kw):
    return (1 + kd) * _PLANE + (kh - 1) * 34 + (kw - 1)


def _quant_body(c_ref, x_ref, qbar_ref, sym_ref):
    x = x_ref[...]
    ds = [jnp.square(x - c_ref[l]) for l in range(_NC)]
    best = ds[0]
    idx = jnp.zeros(x.shape, jnp.int32)
    for l in range(1, _NC):
        cond = ds[l] < best
        idx = jnp.where(cond, jnp.int32(l), idx)
        best = jnp.where(cond, ds[l], best)
    s = jnp.zeros_like(x)
    qs = jnp.zeros_like(x)
    qh = jnp.zeros_like(x)
    for l in range(_NC):
        e = jnp.exp(_SIGMA * (best - ds[l]))
        s = s + e
        qs = qs + e * c_ref[l]
        qh = jnp.where(idx == l, c_ref[l], qh)
    qsoft = qs / s
    qbar_ref[...] = qsoft + (qh - qsoft)
    sym_ref[...] = idx


def _quantize(x, centers):
    n, h, w, c = x.shape
    xf = x.reshape(n * h * w, c)
    qbar, sym = pl.pallas_call(
        _quant_body,
        out_shape=(
            jax.ShapeDtypeStruct(xf.shape, jnp.float32),
            jax.ShapeDtypeStruct(xf.shape, jnp.int32),
        ),
        in_specs=[pl.BlockSpec(memory_space=pltpu.SMEM),
                  pl.BlockSpec(memory_space=pltpu.VMEM)],
        out_specs=(pl.BlockSpec(memory_space=pltpu.VMEM),
                   pl.BlockSpec(memory_space=pltpu.VMEM)),
    )(centers, xf)
    return qbar.reshape(x.shape), sym.reshape(x.shape)


def _pcnn_body(xp_ref, sym_ref, mask_ref, wa_ref, ba_ref, w1_ref, b1_ref,
               w2_ref, b2_ref, wo_ref, bo_ref, bc_ref, ha, r1, xa, xb):
    f32 = jnp.float32
    bf16 = jnp.bfloat16
    mask = mask_ref[...]

    z2 = jnp.zeros((24, 2 * _PLANE), bf16)
    ha[:, : 2 * _PLANE] = z2
    r1[:, : 2 * _PLANE] = z2
    zg = jnp.zeros((24, _PLANE), bf16)
    ha[:, (_NP - 1) * _PLANE:] = zg
    r1[:, (_NP - 1) * _PLANE:] = zg
    xa[13:16, :] = jnp.zeros((3, _NB), bf16)

    def conva(a):
        base = a * _NB
        for ti, (kd, kh, kw) in enumerate(_TAPS_A):
            s = base + _delta(kd, kh, kw)
            xa[pl.ds(ti, 1), :] = xp_ref[0, :, pl.ds(s, _NB)]
        y = jnp.dot(wa_ref[...], xa[...], preferred_element_type=f32)
        blk = jnp.maximum(y + ba_ref[...], 0.0) * mask
        ha[:, pl.ds(base + 2 * _PLANE, _NB)] = blk.astype(bf16)

    def im2col(src, a):
        base = a * _NB
        for ti, (kd, kh, kw) in enumerate(_TAPS_B):
            s = base + _delta(kd, kh, kw)
            xb[pl.ds(24 * ti, 24), :] = src[:, pl.ds(s, _NB)]

    def layer1(a):
        im2col(ha, a)
        y = jnp.dot(w1_ref[...], xb[...], preferred_element_type=f32)
        blk = jnp.maximum(y + b1_ref[...], 0.0) * mask
        r1[:, pl.ds(a * _NB + 2 * _PLANE, _NB)] = blk.astype(bf16)

    def layer2(a):
        im2col(r1, a)
        y = jnp.dot(w2_ref[...], xb[...], preferred_element_type=f32)
        sl = pl.ds(a * _NB + 2 * _PLANE, _NB)
        h = jnp.maximum(ha[:, sl].astype(f32) + y + b2_ref[...], 0.0) * mask
        ha[:, sl] = h.astype(bf16)

    def layero(a):
        im2col(ha, a)
        z = jnp.dot(wo_ref[...], xb[...], preferred_element_type=f32)
        z = z + bo_ref[...]
        m = jnp.max(z, axis=0, keepdims=True)
        lse = m + jnp.log(jnp.sum(jnp.exp(z - m), axis=0, keepdims=True))
        logp = z - lse
        sym = sym_ref[0, :, pl.ds(a * _NB, _NB)]
        bc = jnp.zeros((1, _NB), f32)
        for l in range(_NC):
            bc = jnp.where(sym == l, -logp[l:l + 1, :], bc)
        bc_ref[0, :, pl.ds(a * _NB, _NB)] = bc * _INV_LN2

    for a in range(_NA):
        conva(a)
    for a in range(_NA):
        layer1(a)
    for a in range(_NA):
        layer2(a)
    for a in range(_NA):
        layero(a)


def _pcnn(xp, symf, wam, ba, w1m, b1, w2m, b2, wom, bo):
    n = xp.shape[0]
    pos = jnp.arange(_NB, dtype=jnp.int32) % _PLANE
    hp = pos // 34
    wp = pos % 34
    mask = ((hp >= 1) & (hp < 33) & (wp >= 1) & (wp < 33))
    mask = mask.astype(jnp.float32)[None, :]
    in_specs = [
            pl.BlockSpec((1, 1, _L), lambda i: (i, 0, 0)),
            pl.BlockSpec((1, 1, _D * _PLANE), lambda i: (i, 0, 0)),
            pl.BlockSpec((1, _NB), lambda i: (0, 0)),
            pl.BlockSpec((24, 16), lambda i: (0, 0)),
            pl.BlockSpec((24, 1), lambda i: (0, 0)),
            pl.BlockSpec((24, 336), lambda i: (0, 0)),
            pl.BlockSpec((24, 1), lambda i: (0, 0)),
            pl.BlockSpec((24, 336), lambda i: (0, 0)),
            pl.BlockSpec((24, 1), lambda i: (0, 0)),
            pl.BlockSpec((8, 336), lambda i: (0, 0)),
            pl.BlockSpec((8, 1), lambda i: (0, 0)),
        ]
    return pl.pallas_call(
        _pcnn_body,
        grid=(n,),
        in_specs=in_specs,
        out_specs=pl.BlockSpec((1, 1, _D * _PLANE), lambda i: (i, 0, 0)),
        out_shape=jax.ShapeDtypeStruct((n, 1, _D * _PLANE), jnp.float32),
        scratch_shapes=[
            pltpu.VMEM((24, _L), jnp.bfloat16),
            pltpu.VMEM((24, _L), jnp.bfloat16),
            pltpu.VMEM((16, _NB), jnp.bfloat16),
            pltpu.VMEM((336, _NB), jnp.bfloat16),
        ],
        compiler_params=pltpu.CompilerParams(
            dimension_semantics=("arbitrary",)),
    )(xp, symf, mask, wam, ba, w1m, b1, w2m, b2, wom, bo)


def kernel(inputs, training, centers, wA, bA, w1, b1, w2, b2, wO, bO):
    n = inputs.shape[0]
    qbar, sym = _quantize(inputs, centers)
    c0 = centers[0]

    pc = jnp.transpose(qbar, (0, 3, 1, 2))
    pcp = jnp.pad(pc, ((0, 0), (0, 0), (1, 1), (1, 1)), constant_values=c0)
    padp = jnp.broadcast_to(c0, (n, 1, 34, 34))
    guard = jnp.zeros((n, 1, 34, 34), jnp.float32)
    xp = jnp.concatenate([guard, padp, pcp, guard], axis=1)
    xp = xp.reshape(n, 1, _L).astype(jnp.bfloat16)

    symt = jnp.transpose(sym, (0, 3, 1, 2))
    symp = jnp.pad(symt, ((0, 0), (0, 0), (1, 1), (1, 1)))
    symf = symp.reshape(n, 1, _D * _PLANE)

    wam = jnp.stack([wA[:, 0, kd, kh, kw] for kd, kh, kw in _TAPS_A], axis=1)
    wam = jnp.pad(wam, ((0, 0), (0, 3)))
    w1m = jnp.stack([w1[:, :, kd, kh, kw] for kd, kh, kw in _TAPS_B],
                    axis=1).reshape(24, 336)
    w2m = jnp.stack([w2[:, :, kd, kh, kw] for kd, kh, kw in _TAPS_B],
                    axis=1).reshape(24, 336)
    wom = jnp.stack([wO[:, :, kd, kh, kw] for kd, kh, kw in _TAPS_B],
                    axis=1).reshape(_NC, 336)
    wom = jnp.pad(wom, ((0, 2), (0, 0)))
    bop = jnp.concatenate([bO, jnp.full((2,), -1e30, jnp.float32)])

    bf16 = jnp.bfloat16
    bc = _pcnn(xp, symf, wam.astype(bf16), bA[:, None], w1m.astype(bf16),
               b1[:, None], w2m.astype(bf16), b2[:, None], wom.astype(bf16),
               bop[:, None])
    bc = bc.reshape(n, _D, 34, 34)[:, :, 1:33, 1:33]
    return (qbar, jnp.transpose(bc, (0, 2, 3, 1)))


# speedup vs baseline: 23.1496x; 1.0586x over previous
import jax
import jax.numpy as jnp
import numpy as np
from jax.experimental import pallas as pl
from jax.experimental.pallas import tpu as pltpu

_NC = 6
_SIGMA = 1.0
_D = 96
_PLANE = 34 * 34
_G = 8
_NA = _D // _G
_NB = _G * _PLANE
_NP = _D + 3
_L = _NP * _PLANE

_TAPS_A = [(0, kh, kw) for kh in range(3) for kw in range(3)] + \
          [(1, 0, 0), (1, 0, 1), (1, 0, 2), (1, 1, 0)]
_TAPS_B = _TAPS_A + [(1, 1, 1)]
_BLKS = [(0, 0), (0, 1), (0, 2), (1, 0), (1, 1)]
_INV_LN2 = float(1.0 / np.log(2.0))


def _wkw(w, kw, taps):
    cols = []
    for kd, kh in _BLKS:
        if (kd, kh, kw) in taps:
            cols.append(w[:, :, kd, kh, kw])
        else:
            cols.append(jnp.zeros_like(w[:, :, 0, 0, 0]))
    return jnp.stack(cols, axis=1).reshape(w.shape[0], w.shape[1] * len(_BLKS))


def _delta(kd, kh, kw):
    return (1 + kd) * _PLANE + (kh - 1) * 34 + (kw - 1)


def _quant_body(c_ref, x_ref, qbar_ref, sym_ref):
    x = x_ref[...]
    ds = [jnp.square(x - c_ref[l]) for l in range(_NC)]
    best = ds[0]
    idx = jnp.zeros(x.shape, jnp.int32)
    for l in range(1, _NC):
        cond = ds[l] < best
        idx = jnp.where(cond, jnp.int32(l), idx)
        best = jnp.where(cond, ds[l], best)
    s = jnp.zeros_like(x)
    qs = jnp.zeros_like(x)
    qh = jnp.zeros_like(x)
    for l in range(_NC):
        e = jnp.exp(_SIGMA * (best - ds[l]))
        s = s + e
        qs = qs + e * c_ref[l]
        qh = jnp.where(idx == l, c_ref[l], qh)
    qsoft = qs / s
    qbar_ref[...] = qsoft + (qh - qsoft)
    sym_ref[...] = idx


def _quantize(x, centers):
    n, h, w, c = x.shape
    xf = x.reshape(n * h * w, c)
    qbar, sym = pl.pallas_call(
        _quant_body,
        out_shape=(
            jax.ShapeDtypeStruct(xf.shape, jnp.float32),
            jax.ShapeDtypeStruct(xf.shape, jnp.int32),
        ),
        in_specs=[pl.BlockSpec(memory_space=pltpu.SMEM),
                  pl.BlockSpec(memory_space=pltpu.VMEM)],
        out_specs=(pl.BlockSpec(memory_space=pltpu.VMEM),
                   pl.BlockSpec(memory_space=pltpu.VMEM)),
    )(centers, xf)
    return qbar.reshape(x.shape), sym.reshape(x.shape)


def _pcnn_body(xp_ref, sym_ref, mask_ref, wa_ref, ba_ref, w1_ref, b1_ref,
               w2_ref, b2_ref, wo_ref, bo_ref, bc_ref, ha, r1, xa, xb):
    f32 = jnp.float32
    bf16 = jnp.bfloat16
    mask = mask_ref[...]

    z2 = jnp.zeros((24, 2 * _PLANE), bf16)
    ha[:, : 2 * _PLANE] = z2
    r1[:, : 2 * _PLANE] = z2
    zg = jnp.zeros((24, _PLANE), bf16)
    ha[:, (_NP - 1) * _PLANE:] = zg
    r1[:, (_NP - 1) * _PLANE:] = zg
    xa[5:8, :] = jnp.zeros((3, _NB + 2), bf16)

    def conva(a):
        base = a * _NB
        for bi, (kd, kh) in enumerate(_BLKS):
            s = base + (1 + kd) * _PLANE + (kh - 1) * 34 - 1
            xa[pl.ds(bi, 1), :] = xp_ref[0, :, pl.ds(s, _NB + 2)]
        y = jnp.dot(wa_ref[:, 0:8], xa[:, 0:_NB], preferred_element_type=f32)
        y += jnp.dot(wa_ref[:, 8:16], xa[:, 1:_NB + 1],
                     preferred_element_type=f32)
        y += jnp.dot(wa_ref[:, 16:24], xa[:, 2:_NB + 2],
                     preferred_element_type=f32)
        blk = jnp.maximum(y + ba_ref[...], 0.0) * mask
        ha[:, pl.ds(base + 2 * _PLANE, _NB)] = blk.astype(bf16)

    def im2col(src, a):
        base = a * _NB
        for bi, (kd, kh) in enumerate(_BLKS):
            s = base + (1 + kd) * _PLANE + (kh - 1) * 34 - 1
            xb[pl.ds(24 * bi, 24), :] = src[:, pl.ds(s, _NB + 2)]

    def mm3(w_ref):
        y = jnp.dot(w_ref[:, 0:120], xb[:, 0:_NB], preferred_element_type=f32)
        y += jnp.dot(w_ref[:, 120:240], xb[:, 1:_NB + 1],
                     preferred_element_type=f32)
        y += jnp.dot(w_ref[:, 240:360], xb[:, 2:_NB + 2],
                     preferred_element_type=f32)
        return y

    def layer1(a):
        im2col(ha, a)
        y = mm3(w1_ref)
        blk = jnp.maximum(y + b1_ref[...], 0.0) * mask
        r1[:, pl.ds(a * _NB + 2 * _PLANE, _NB)] = blk.astype(bf16)

    def layer2(a):
        im2col(r1, a)
        y = mm3(w2_ref)
        sl = pl.ds(a * _NB + 2 * _PLANE, _NB)
        h = jnp.maximum(ha[:, sl].astype(f32) + y + b2_ref[...], 0.0) * mask
        ha[:, sl] = h.astype(bf16)

    def layero(a):
        im2col(ha, a)
        z = mm3(wo_ref)
        z = z + bo_ref[...]
        m = jnp.max(z, axis=0, keepdims=True)
        lse = m + jnp.log(jnp.sum(jnp.exp(z - m), axis=0, keepdims=True))
        logp = z - lse
        sym = sym_ref[0, :, pl.ds(a * _NB, _NB)]
        bc = jnp.zeros((1, _NB), f32)
        for l in range(_NC):
            bc = jnp.where(sym == l, -logp[l:l + 1, :], bc)
        bc_ref[0, :, pl.ds(a * _NB, _NB)] = bc * _INV_LN2

    for a in range(_NA):
        conva(a)
    for a in range(_NA):
        layer1(a)
    for a in range(_NA):
        layer2(a)
    for a in range(_NA):
        layero(a)


def _pcnn(xp, symf, wam, ba, w1m, b1, w2m, b2, wom, bo):
    n = xp.shape[0]
    pos = jnp.arange(_NB, dtype=jnp.int32) % _PLANE
    hp = pos // 34
    wp = pos % 34
    mask = ((hp >= 1) & (hp < 33) & (wp >= 1) & (wp < 33))
    mask = mask.astype(jnp.float32)[None, :]
    in_specs = [
            pl.BlockSpec((1, 1, _L), lambda i: (i, 0, 0)),
            pl.BlockSpec((1, 1, _D * _PLANE), lambda i: (i, 0, 0)),
            pl.BlockSpec((1, _NB), lambda i: (0, 0)),
            pl.BlockSpec((24, 24), lambda i: (0, 0)),
            pl.BlockSpec((24, 1), lambda i: (0, 0)),
            pl.BlockSpec((24, 360), lambda i: (0, 0)),
            pl.BlockSpec((24, 1), lambda i: (0, 0)),
            pl.BlockSpec((24, 360), lambda i: (0, 0)),
            pl.BlockSpec((24, 1), lambda i: (0, 0)),
            pl.BlockSpec((8, 360), lambda i: (0, 0)),
            pl.BlockSpec((8, 1), lambda i: (0, 0)),
        ]
    return pl.pallas_call(
        _pcnn_body,
        grid=(n,),
        in_specs=in_specs,
        out_specs=pl.BlockSpec((1, 1, _D * _PLANE), lambda i: (i, 0, 0)),
        out_shape=jax.ShapeDtypeStruct((n, 1, _D * _PLANE), jnp.float32),
        scratch_shapes=[
            pltpu.VMEM((24, _L), jnp.bfloat16),
            pltpu.VMEM((24, _L), jnp.bfloat16),
            pltpu.VMEM((8, _NB + 2), jnp.bfloat16),
            pltpu.VMEM((120, _NB + 2), jnp.bfloat16),
        ],
        compiler_params=pltpu.CompilerParams(
            dimension_semantics=("arbitrary",)),
    )(xp, symf, mask, wam, ba, w1m, b1, w2m, b2, wom, bo)


def kernel(inputs, training, centers, wA, bA, w1, b1, w2, b2, wO, bO):
    n = inputs.shape[0]
    qbar, sym = _quantize(inputs, centers)
    c0 = centers[0]

    pc = jnp.transpose(qbar, (0, 3, 1, 2))
    pcp = jnp.pad(pc, ((0, 0), (0, 0), (1, 1), (1, 1)), constant_values=c0)
    padp = jnp.broadcast_to(c0, (n, 1, 34, 34))
    guard = jnp.zeros((n, 1, 34, 34), jnp.float32)
    xp = jnp.concatenate([guard, padp, pcp, guard], axis=1)
    xp = xp.reshape(n, 1, _L).astype(jnp.bfloat16)

    symt = jnp.transpose(sym, (0, 3, 1, 2))
    symp = jnp.pad(symt, ((0, 0), (0, 0), (1, 1), (1, 1)))
    symf = symp.reshape(n, 1, _D * _PLANE)

    wam = jnp.concatenate(
        [jnp.pad(_wkw(wA, kw, _TAPS_A), ((0, 0), (0, 3)))
         for kw in range(3)], axis=1)
    w1m = jnp.concatenate([_wkw(w1, kw, _TAPS_B) for kw in range(3)], axis=1)
    w2m = jnp.concatenate([_wkw(w2, kw, _TAPS_B) for kw in range(3)], axis=1)
    wom = jnp.concatenate([_wkw(wO, kw, _TAPS_B) for kw in range(3)], axis=1)
    wom = jnp.pad(wom, ((0, 2), (0, 0)))
    bop = jnp.concatenate([bO, jnp.full((2,), -1e30, jnp.float32)])

    bf16 = jnp.bfloat16
    bc = _pcnn(xp, symf, wam.astype(bf16), bA[:, None], w1m.astype(bf16),
               b1[:, None], w2m.astype(bf16), b2[:, None], wom.astype(bf16),
               bop[:, None])
    bc = bc.reshape(n, _D, 34, 34)[:, :, 1:33, 1:33]
    return (qbar, jnp.transpose(bc, (0, 2, 3, 1)))


# speedup vs baseline: 24.8083x; 1.0717x over previous
import jax
import jax.numpy as jnp
import numpy as np
from jax.experimental import pallas as pl
from jax.experimental.pallas import tpu as pltpu

_NC = 6
_SIGMA = 1.0
_D = 96
_PLANE = 34 * 34
_G = 16
_NA = _D // _G
_NB = _G * _PLANE
_NP = _D + 3
_L = _NP * _PLANE

_TAPS_A = [(0, kh, kw) for kh in range(3) for kw in range(3)] + \
          [(1, 0, 0), (1, 0, 1), (1, 0, 2), (1, 1, 0)]
_TAPS_B = _TAPS_A + [(1, 1, 1)]
_BLKS = [(0, 0), (0, 1), (0, 2), (1, 0), (1, 1)]
_INV_LN2 = float(1.0 / np.log(2.0))


def _wkw(w, kw, taps):
    cols = []
    for kd, kh in _BLKS:
        if (kd, kh, kw) in taps:
            cols.append(w[:, :, kd, kh, kw])
        else:
            cols.append(jnp.zeros_like(w[:, :, 0, 0, 0]))
    return jnp.stack(cols, axis=1).reshape(w.shape[0], w.shape[1] * len(_BLKS))


def _delta(kd, kh, kw):
    return (1 + kd) * _PLANE + (kh - 1) * 34 + (kw - 1)


def _quant_body(c_ref, x_ref, qbar_ref, sym_ref):
    x = x_ref[...]
    ds = [jnp.square(x - c_ref[l]) for l in range(_NC)]
    best = ds[0]
    idx = jnp.zeros(x.shape, jnp.int32)
    for l in range(1, _NC):
        cond = ds[l] < best
        idx = jnp.where(cond, jnp.int32(l), idx)
        best = jnp.where(cond, ds[l], best)
    s = jnp.zeros_like(x)
    qs = jnp.zeros_like(x)
    qh = jnp.zeros_like(x)
    for l in range(_NC):
        e = jnp.exp(_SIGMA * (best - ds[l]))
        s = s + e
        qs = qs + e * c_ref[l]
        qh = jnp.where(idx == l, c_ref[l], qh)
    qsoft = qs / s
    qbar_ref[...] = qsoft + (qh - qsoft)
    sym_ref[...] = idx


def _quantize(x, centers):
    n, h, w, c = x.shape
    xf = x.reshape(n * h * w, c)
    qbar, sym = pl.pallas_call(
        _quant_body,
        out_shape=(
            jax.ShapeDtypeStruct(xf.shape, jnp.float32),
            jax.ShapeDtypeStruct(xf.shape, jnp.int32),
        ),
        in_specs=[pl.BlockSpec(memory_space=pltpu.SMEM),
                  pl.BlockSpec(memory_space=pltpu.VMEM)],
        out_specs=(pl.BlockSpec(memory_space=pltpu.VMEM),
                   pl.BlockSpec(memory_space=pltpu.VMEM)),
    )(centers, xf)
    return qbar.reshape(x.shape), sym.reshape(x.shape)


def _pcnn_body(xp_ref, sym_ref, mask_ref, wa_ref, ba_ref, w1_ref, b1_ref,
               w2_ref, b2_ref, wo_ref, bo_ref, bc_ref, ha, r1, xa, xb):
    f32 = jnp.float32
    bf16 = jnp.bfloat16
    mask = mask_ref[...]

    z2 = jnp.zeros((24, 2 * _PLANE), bf16)
    ha[:, : 2 * _PLANE] = z2
    r1[:, : 2 * _PLANE] = z2
    zg = jnp.zeros((24, _PLANE), bf16)
    ha[:, (_NP - 1) * _PLANE:] = zg
    r1[:, (_NP - 1) * _PLANE:] = zg
    xa[5:8, :] = jnp.zeros((3, _NB + 2), bf16)

    def conva(a):
        base = a * _NB
        for bi, (kd, kh) in enumerate(_BLKS):
            s = base + (1 + kd) * _PLANE + (kh - 1) * 34 - 1
            xa[pl.ds(bi, 1), :] = xp_ref[0, :, pl.ds(s, _NB + 2)]
        y = jnp.dot(wa_ref[:, 0:8], xa[:, 0:_NB], preferred_element_type=f32)
        y += jnp.dot(wa_ref[:, 8:16], xa[:, 1:_NB + 1],
                     preferred_element_type=f32)
        y += jnp.dot(wa_ref[:, 16:24], xa[:, 2:_NB + 2],
                     preferred_element_type=f32)
        blk = jnp.maximum(y + ba_ref[...], 0.0) * mask
        ha[:, pl.ds(base + 2 * _PLANE, _NB)] = blk.astype(bf16)

    def im2col(src, a):
        base = a * _NB
        for bi, (kd, kh) in enumerate(_BLKS):
            s = base + (1 + kd) * _PLANE + (kh - 1) * 34 - 1
            xb[pl.ds(24 * bi, 24), :] = src[:, pl.ds(s, _NB + 2)]

    def mm3(w_ref):
        y = jnp.dot(w_ref[:, 0:120], xb[:, 0:_NB], preferred_element_type=f32)
        y += jnp.dot(w_ref[:, 120:240], xb[:, 1:_NB + 1],
                     preferred_element_type=f32)
        y += jnp.dot(w_ref[:, 240:360], xb[:, 2:_NB + 2],
                     preferred_element_type=f32)
        return y

    def layer1(a):
        im2col(ha, a)
        y = mm3(w1_ref)
        blk = jnp.maximum(y + b1_ref[...], 0.0) * mask
        r1[:, pl.ds(a * _NB + 2 * _PLANE, _NB)] = blk.astype(bf16)

    def layer2(a):
        im2col(r1, a)
        y = mm3(w2_ref)
        sl = pl.ds(a * _NB + 2 * _PLANE, _NB)
        h = jnp.maximum(ha[:, sl].astype(f32) + y + b2_ref[...], 0.0) * mask
        ha[:, sl] = h.astype(bf16)

    def layero(a):
        im2col(ha, a)
        z = mm3(wo_ref)
        z = z + bo_ref[...]
        m = jnp.max(z, axis=0, keepdims=True)
        lse = m + jnp.log(jnp.sum(jnp.exp(z - m), axis=0, keepdims=True))
        logp = z - lse
        sym = sym_ref[0, :, pl.ds(a * _NB, _NB)]
        bc = jnp.zeros((1, _NB), f32)
        for l in range(_NC):
            bc = jnp.where(sym == l, -logp[l:l + 1, :], bc)
        bc_ref[0, :, pl.ds(a * _NB, _NB)] = bc * _INV_LN2

    for a in range(_NA):
        conva(a)
    for a in range(_NA):
        layer1(a)
    for a in range(_NA):
        layer2(a)
    for a in range(_NA):
        layero(a)


def _pcnn(xp, symf, wam, ba, w1m, b1, w2m, b2, wom, bo):
    n = xp.shape[0]
    pos = jnp.arange(_NB, dtype=jnp.int32) % _PLANE
    hp = pos // 34
    wp = pos % 34
    mask = ((hp >= 1) & (hp < 33) & (wp >= 1) & (wp < 33))
    mask = mask.astype(jnp.float32)[None, :]
    in_specs = [
            pl.BlockSpec((1, 1, _L), lambda i: (i, 0, 0)),
            pl.BlockSpec((1, 1, _D * _PLANE), lambda i: (i, 0, 0)),
            pl.BlockSpec((1, _NB), lambda i: (0, 0)),
            pl.BlockSpec((24, 24), lambda i: (0, 0)),
            pl.BlockSpec((24, 1), lambda i: (0, 0)),
            pl.BlockSpec((24, 360), lambda i: (0, 0)),
            pl.BlockSpec((24, 1), lambda i: (0, 0)),
            pl.BlockSpec((24, 360), lambda i: (0, 0)),
            pl.BlockSpec((24, 1), lambda i: (0, 0)),
            pl.BlockSpec((8, 360), lambda i: (0, 0)),
            pl.BlockSpec((8, 1), lambda i: (0, 0)),
        ]
    return pl.pallas_call(
        _pcnn_body,
        grid=(n,),
        in_specs=in_specs,
        out_specs=pl.BlockSpec((1, 1, _D * _PLANE), lambda i: (i, 0, 0)),
        out_shape=jax.ShapeDtypeStruct((n, 1, _D * _PLANE), jnp.float32),
        scratch_shapes=[
            pltpu.VMEM((24, _L), jnp.bfloat16),
            pltpu.VMEM((24, _L), jnp.bfloat16),
            pltpu.VMEM((8, _NB + 2), jnp.bfloat16),
            pltpu.VMEM((120, _NB + 2), jnp.bfloat16),
        ],
        compiler_params=pltpu.CompilerParams(
            dimension_semantics=("arbitrary",)),
    )(xp, symf, mask, wam, ba, w1m, b1, w2m, b2, wom, bo)


def kernel(inputs, training, centers, wA, bA, w1, b1, w2, b2, wO, bO):
    n = inputs.shape[0]
    qbar, sym = _quantize(inputs, centers)
    c0 = centers[0]

    pc = jnp.transpose(qbar, (0, 3, 1, 2))
    pcp = jnp.pad(pc, ((0, 0), (0, 0), (1, 1), (1, 1)), constant_values=c0)
    padp = jnp.broadcast_to(c0, (n, 1, 34, 34))
    guard = jnp.zeros((n, 1, 34, 34), jnp.float32)
    xp = jnp.concatenate([guard, padp, pcp, guard], axis=1)
    xp = xp.reshape(n, 1, _L).astype(jnp.bfloat16)

    symt = jnp.transpose(sym, (0, 3, 1, 2))
    symp = jnp.pad(symt, ((0, 0), (0, 0), (1, 1), (1, 1)))
    symf = symp.reshape(n, 1, _D * _PLANE)

    wam = jnp.concatenate(
        [jnp.pad(_wkw(wA, kw, _TAPS_A), ((0, 0), (0, 3)))
         for kw in range(3)], axis=1)
    w1m = jnp.concatenate([_wkw(w1, kw, _TAPS_B) for kw in range(3)], axis=1)
    w2m = jnp.concatenate([_wkw(w2, kw, _TAPS_B) for kw in range(3)], axis=1)
    wom = jnp.concatenate([_wkw(wO, kw, _TAPS_B) for kw in range(3)], axis=1)
    wom = jnp.pad(wom, ((0, 2), (0, 0)))
    bop = jnp.concatenate([bO, jnp.full((2,), -1e30, jnp.float32)])

    bf16 = jnp.bfloat16
    bc = _pcnn(xp, symf, wam.astype(bf16), bA[:, None], w1m.astype(bf16),
               b1[:, None], w2m.astype(bf16), b2[:, None], wom.astype(bf16),
               bop[:, None])
    bc = bc.reshape(n, _D, 34, 34)[:, :, 1:33, 1:33]
    return (qbar, jnp.transpose(bc, (0, 2, 3, 1)))


# speedup vs baseline: 25.2297x; 1.0170x over previous
import jax
import jax.numpy as jnp
import numpy as np
from jax.experimental import pallas as pl
from jax.experimental.pallas import tpu as pltpu

_NC = 6
_SIGMA = 1.0
_D = 96
_PLANE = 34 * 34
_G = 32
_NA = _D // _G
_NB = _G * _PLANE
_NP = _D + 3
_L = _NP * _PLANE

_TAPS_A = [(0, kh, kw) for kh in range(3) for kw in range(3)] + \
          [(1, 0, 0), (1, 0, 1), (1, 0, 2), (1, 1, 0)]
_TAPS_B = _TAPS_A + [(1, 1, 1)]
_BLKS = [(0, 0), (0, 1), (0, 2), (1, 0), (1, 1)]
_INV_LN2 = float(1.0 / np.log(2.0))


def _wkw(w, kw, taps):
    cols = []
    for kd, kh in _BLKS:
        if (kd, kh, kw) in taps:
            cols.append(w[:, :, kd, kh, kw])
        else:
            cols.append(jnp.zeros_like(w[:, :, 0, 0, 0]))
    return jnp.stack(cols, axis=1).reshape(w.shape[0], w.shape[1] * len(_BLKS))


def _delta(kd, kh, kw):
    return (1 + kd) * _PLANE + (kh - 1) * 34 + (kw - 1)


def _quant_body(c_ref, x_ref, qbar_ref, sym_ref):
    x = x_ref[...]
    ds = [jnp.square(x - c_ref[l]) for l in range(_NC)]
    best = ds[0]
    idx = jnp.zeros(x.shape, jnp.int32)
    for l in range(1, _NC):
        cond = ds[l] < best
        idx = jnp.where(cond, jnp.int32(l), idx)
        best = jnp.where(cond, ds[l], best)
    s = jnp.zeros_like(x)
    qs = jnp.zeros_like(x)
    qh = jnp.zeros_like(x)
    for l in range(_NC):
        e = jnp.exp(_SIGMA * (best - ds[l]))
        s = s + e
        qs = qs + e * c_ref[l]
        qh = jnp.where(idx == l, c_ref[l], qh)
    qsoft = qs / s
    qbar_ref[...] = qsoft + (qh - qsoft)
    sym_ref[...] = idx


def _quantize(x, centers):
    n, h, w, c = x.shape
    xf = x.reshape(n * h * w, c)
    qbar, sym = pl.pallas_call(
        _quant_body,
        out_shape=(
            jax.ShapeDtypeStruct(xf.shape, jnp.float32),
            jax.ShapeDtypeStruct(xf.shape, jnp.int32),
        ),
        in_specs=[pl.BlockSpec(memory_space=pltpu.SMEM),
                  pl.BlockSpec(memory_space=pltpu.VMEM)],
        out_specs=(pl.BlockSpec(memory_space=pltpu.VMEM),
                   pl.BlockSpec(memory_space=pltpu.VMEM)),
    )(centers, xf)
    return qbar.reshape(x.shape), sym.reshape(x.shape)


def _pcnn_body(xp_ref, sym_ref, mask_ref, wa_ref, ba_ref, w1_ref, b1_ref,
               w2_ref, b2_ref, wo_ref, bo_ref, bc_ref, ha, r1, xa, xb):
    f32 = jnp.float32
    bf16 = jnp.bfloat16
    mask = mask_ref[...]

    z2 = jnp.zeros((24, 2 * _PLANE), bf16)
    ha[:, : 2 * _PLANE] = z2
    r1[:, : 2 * _PLANE] = z2
    zg = jnp.zeros((24, _PLANE), bf16)
    ha[:, (_NP - 1) * _PLANE:] = zg
    r1[:, (_NP - 1) * _PLANE:] = zg
    xa[5:8, :] = jnp.zeros((3, _NB + 2), bf16)

    def conva(a):
        base = a * _NB
        for bi, (kd, kh) in enumerate(_BLKS):
            s = base + (1 + kd) * _PLANE + (kh - 1) * 34 - 1
            xa[pl.ds(bi, 1), :] = xp_ref[0, :, pl.ds(s, _NB + 2)]
        y = jnp.dot(wa_ref[:, 0:8], xa[:, 0:_NB], preferred_element_type=f32)
        y += jnp.dot(wa_ref[:, 8:16], xa[:, 1:_NB + 1],
                     preferred_element_type=f32)
        y += jnp.dot(wa_ref[:, 16:24], xa[:, 2:_NB + 2],
                     preferred_element_type=f32)
        blk = jnp.maximum(y + ba_ref[...], 0.0) * mask
        ha[:, pl.ds(base + 2 * _PLANE, _NB)] = blk.astype(bf16)

    def im2col(src, a):
        base = a * _NB
        for bi, (kd, kh) in enumerate(_BLKS):
            s = base + (1 + kd) * _PLANE + (kh - 1) * 34 - 1
            xb[pl.ds(24 * bi, 24), :] = src[:, pl.ds(s, _NB + 2)]

    def mm3(w_ref):
        y = jnp.dot(w_ref[:, 0:120], xb[:, 0:_NB], preferred_element_type=f32)
        y += jnp.dot(w_ref[:, 120:240], xb[:, 1:_NB + 1],
                     preferred_element_type=f32)
        y += jnp.dot(w_ref[:, 240:360], xb[:, 2:_NB + 2],
                     preferred_element_type=f32)
        return y

    def layer1(a):
        im2col(ha, a)
        y = mm3(w1_ref)
        blk = jnp.maximum(y + b1_ref[...], 0.0) * mask
        r1[:, pl.ds(a * _NB + 2 * _PLANE, _NB)] = blk.astype(bf16)

    def layer2(a):
        im2col(r1, a)
        y = mm3(w2_ref)
        sl = pl.ds(a * _NB + 2 * _PLANE, _NB)
        h = jnp.maximum(ha[:, sl].astype(f32) + y + b2_ref[...], 0.0) * mask
        ha[:, sl] = h.astype(bf16)

    def layero(a):
        im2col(ha, a)
        z = mm3(wo_ref)
        z = z + bo_ref[...]
        m = jnp.max(z, axis=0, keepdims=True)
        lse = m + jnp.log(jnp.sum(jnp.exp(z - m), axis=0, keepdims=True))
        logp = z - lse
        sym = sym_ref[0, :, pl.ds(a * _NB, _NB)]
        bc = jnp.zeros((1, _NB), f32)
        for l in range(_NC):
            bc = jnp.where(sym == l, -logp[l:l + 1, :], bc)
        bc_ref[0, :, pl.ds(a * _NB, _NB)] = bc * _INV_LN2

    for a in range(_NA):
        conva(a)
    for a in range(_NA):
        layer1(a)
    for a in range(_NA):
        layer2(a)
    for a in range(_NA):
        layero(a)


def _pcnn(xp, symf, wam, ba, w1m, b1, w2m, b2, wom, bo):
    n = xp.shape[0]
    pos = jnp.arange(_NB, dtype=jnp.int32) % _PLANE
    hp = pos // 34
    wp = pos % 34
    mask = ((hp >= 1) & (hp < 33) & (wp >= 1) & (wp < 33))
    mask = mask.astype(jnp.float32)[None, :]
    in_specs = [
            pl.BlockSpec((1, 1, _L), lambda i: (i, 0, 0)),
            pl.BlockSpec((1, 1, _D * _PLANE), lambda i: (i, 0, 0)),
            pl.BlockSpec((1, _NB), lambda i: (0, 0)),
            pl.BlockSpec((24, 24), lambda i: (0, 0)),
            pl.BlockSpec((24, 1), lambda i: (0, 0)),
            pl.BlockSpec((24, 360), lambda i: (0, 0)),
            pl.BlockSpec((24, 1), lambda i: (0, 0)),
            pl.BlockSpec((24, 360), lambda i: (0, 0)),
            pl.BlockSpec((24, 1), lambda i: (0, 0)),
            pl.BlockSpec((8, 360), lambda i: (0, 0)),
            pl.BlockSpec((8, 1), lambda i: (0, 0)),
        ]
    return pl.pallas_call(
        _pcnn_body,
        grid=(n,),
        in_specs=in_specs,
        out_specs=pl.BlockSpec((1, 1, _D * _PLANE), lambda i: (i, 0, 0)),
        out_shape=jax.ShapeDtypeStruct((n, 1, _D * _PLANE), jnp.float32),
        scratch_shapes=[
            pltpu.VMEM((24, _L), jnp.bfloat16),
            pltpu.VMEM((24, _L), jnp.bfloat16),
            pltpu.VMEM((8, _NB + 2), jnp.bfloat16),
            pltpu.VMEM((120, _NB + 2), jnp.bfloat16),
        ],
        compiler_params=pltpu.CompilerParams(
            dimension_semantics=("arbitrary",)),
    )(xp, symf, mask, wam, ba, w1m, b1, w2m, b2, wom, bo)


def kernel(inputs, training, centers, wA, bA, w1, b1, w2, b2, wO, bO):
    n = inputs.shape[0]
    qbar, sym = _quantize(inputs, centers)
    c0 = centers[0]

    pc = jnp.transpose(qbar, (0, 3, 1, 2))
    pcp = jnp.pad(pc, ((0, 0), (0, 0), (1, 1), (1, 1)), constant_values=c0)
    padp = jnp.broadcast_to(c0, (n, 1, 34, 34))
    guard = jnp.zeros((n, 1, 34, 34), jnp.float32)
    xp = jnp.concatenate([guard, padp, pcp, guard], axis=1)
    xp = xp.reshape(n, 1, _L).astype(jnp.bfloat16)

    symt = jnp.transpose(sym, (0, 3, 1, 2))
    symp = jnp.pad(symt, ((0, 0), (0, 0), (1, 1), (1, 1)))
    symf = symp.reshape(n, 1, _D * _PLANE)

    wam = jnp.concatenate(
        [jnp.pad(_wkw(wA, kw, _TAPS_A), ((0, 0), (0, 3)))
         for kw in range(3)], axis=1)
    w1m = jnp.concatenate([_wkw(w1, kw, _TAPS_B) for kw in range(3)], axis=1)
    w2m = jnp.concatenate([_wkw(w2, kw, _TAPS_B) for kw in range(3)], axis=1)
    wom = jnp.concatenate([_wkw(wO, kw, _TAPS_B) for kw in range(3)], axis=1)
    wom = jnp.pad(wom, ((0, 2), (0, 0)))
    bop = jnp.concatenate([bO, jnp.full((2,), -1e30, jnp.float32)])

    bf16 = jnp.bfloat16
    bc = _pcnn(xp, symf, wam.astype(bf16), bA[:, None], w1m.astype(bf16),
               b1[:, None], w2m.astype(bf16), b2[:, None], wom.astype(bf16),
               bop[:, None])
    bc = bc.reshape(n, _D, 34, 34)[:, :, 1:33, 1:33]
    return (qbar, jnp.transpose(bc, (0, 2, 3, 1)))
